# baseline (speedup 1.0000x reference)
import functools

import numpy as np

import jax
import jax.numpy as jnp
from jax import lax
from jax.experimental import pallas as pl
from jax.experimental.pallas import tpu as pltpu
from jax.experimental.pallas import tpu_sc as plsc

N = 10000
E = 320000
H = 4
D = 32
DIM = 128
HD = H * D

NC = 2
NS = 16
CH = 512
NCH = E // CH
CPB = -(-NCH // NS)
RB = 624
ZR = 48

_f32 = jnp.float32
_i32 = jnp.int32


def _prep_body(x_ref, w_ref, alr_ref, feat_ref, elr_ref):
    x = x_ref[0]
    feat = jnp.dot(x, w_ref[0], preferred_element_type=_f32)
    elr = jnp.dot(feat, alr_ref[0], preferred_element_type=_f32)
    feat_ref[...] = feat
    elr_ref[...] = elr[None]


def _prep(xs, Ws, Alrs):
    return pl.pallas_call(
        _prep_body,
        grid=(2, 10),
        in_specs=[
            pl.BlockSpec((1, 1000, DIM), lambda l, i: (l, i, 0)),
            pl.BlockSpec((1, DIM, HD), lambda l, i: (l, 0, 0)),
            pl.BlockSpec((1, HD, 2 * H), lambda l, i: (l, 0, 0)),
        ],
        out_specs=[
            pl.BlockSpec((1000, HD), lambda l, i: (l * 10 + i, 0)),
            pl.BlockSpec((1, 1000, 2 * H), lambda l, i: (l, i, 0)),
        ],
        out_shape=[
            jax.ShapeDtypeStruct((2 * N, HD), _f32),
            jax.ShapeDtypeStruct((2, N, 2 * H), _f32),
        ],
    )(xs, Ws, Alrs)


def _sc_mesh():
    return plsc.VectorSubcoreMesh(core_axis_name="c", subcore_axis_name="s")


@functools.partial(
    pl.kernel,
    out_type=(
        jax.ShapeDtypeStruct((2, NCH, H, CH), _f32),
        jax.ShapeDtypeStruct((2 * NS * N * H,), _f32),
    ),
    mesh=_sc_mesh(),
    scratch_types=[
        pltpu.VMEM((N * 2 * H,), _f32),
        pltpu.VMEM((N * H,), _f32),
        pltpu.VMEM((2, 2, CH), _i32),
        pltpu.VMEM((2, H, CH), _f32),
        pltpu.SemaphoreType.DMA,
        pltpu.SemaphoreType.DMA,
        pltpu.SemaphoreType.DMA,
        pltpu.SemaphoreType.DMA,
    ],
    compiler_params=pltpu.CompilerParams(needs_layout_passes=False),
)
def _phase1(elr_hbm, edc_hbm, ex_hbm, part_hbm, elr_v, den_v, edc_v, ex_v,
            iesem0, iesem1, oxsem0, oxsem1):
    c = lax.axis_index("c")
    s = lax.axis_index("s")
    zeros16 = jnp.zeros((16,), _f32)
    iesems = (iesem0, iesem1)
    oxsems = (oxsem0, oxsem1)

    pltpu.sync_copy(elr_hbm.at[pl.ds(c * (N * 2 * H), N * 2 * H)], elr_v)

    @plsc.parallel_loop(0, (N * H) // 16, unroll=8)
    def zb(i):
        den_v[pl.ds(i * 16, 16)] = zeros16

    def fire_in(ci, b):
        ch = s + ci * NS

        @pl.when(ch < NCH)
        def _():
            pltpu.async_copy(edc_hbm.at[c, ch], edc_v.at[b], iesems[b])

    def process(ci, b):
        ch = s + ci * NS

        @pl.when(ch < NCH)
        def _():
            pltpu.make_async_copy(
                edc_hbm.at[c, ch], edc_v.at[b], iesems[b]
            ).wait()

            @plsc.parallel_loop(0, CH // 16, unroll=4)
            def grp(g):
                src16 = edc_v[b, 0, pl.ds(g * 16, 16)]
                dst16 = edc_v[b, 1, pl.ds(g * 16, 16)]
                for h in range(H):
                    el = plsc.load_gather(elr_v, [src16 * (2 * H) + h])
                    er = plsc.load_gather(elr_v, [dst16 * (2 * H) + (H + h)])
                    e = el + er
                    e = jnp.where(e >= 0, e, 0.2 * e)
                    ex = jnp.exp(e)
                    ex_v[b, h, pl.ds(g * 16, 16)] = ex
                    plsc.addupdate_scatter(den_v, [dst16 * H + h], ex)
            pltpu.async_copy(ex_v.at[b], ex_hbm.at[c, ch], oxsems[b])

    def drain_out(ci, b):
        ch = s + ci * NS

        @pl.when(ch < NCH)
        def _():
            pltpu.make_async_copy(
                ex_v.at[b], ex_hbm.at[c, ch], oxsems[b]
            ).wait()

    fire_in(0, 0)

    def pair_body(i2, carry):
        for b in (0, 1):
            ci = i2 * 2 + b
            fire_in(ci + 1, 1 - b)

            @pl.when(ci >= 2)
            def _():
                drain_out(ci - 2, b)

            process(ci, b)
        return carry

    lax.fori_loop(0, (CPB + 2) // 2, pair_body, 0)
    pltpu.sync_copy(den_v, part_hbm.at[pl.ds((c * NS + s) * (N * H), N * H)])


@functools.partial(
    pl.kernel,
    out_type=(
        jax.ShapeDtypeStruct((E * H,), _f32),
        jax.ShapeDtypeStruct((E * H,), _f32),
    ),
    mesh=_sc_mesh(),
    scratch_types=[
        pltpu.VMEM((N * H,), _f32),
        pltpu.VMEM((NS * 2560,), _f32),
        pltpu.VMEM((2, 2, CH), _i32),
        pltpu.VMEM((2, H, CH), _f32),
        pltpu.VMEM((2 * CH * H,), _f32),
        pltpu.VMEM_SHARED((N * H,), _f32),
        pltpu.SemaphoreType.DMA,
        pltpu.SemaphoreType.DMA,
        pltpu.SemaphoreType.DMA,
        pltpu.SemaphoreType.DMA,
        pltpu.SemaphoreType.DMA,
    ],
    compiler_params=pltpu.CompilerParams(needs_layout_passes=False),
)
def _phase2a(edc_hbm, ex_hbm, part_hbm, alpha0_hbm, alpha1_hbm,
             invden_v, red_v, edc_v, ex_v, alpha_v, inv_sp,
             isem0, isem1, oasem0, oasem1, rsem):
    c = lax.axis_index("c")
    s = lax.axis_index("s")
    lanes = lax.iota(_i32, 16)
    isems = (isem0, isem1)
    oasems = (oasem0, oasem1)

    def _reduce(off, L):
        for p in range(NS):
            pltpu.async_copy(
                part_hbm.at[pl.ds((c * NS + p) * (N * H) + off, L)],
                red_v.at[pl.ds(p * 2560, L)], rsem,
            )
        for p in range(NS):
            pltpu.make_async_copy(
                part_hbm.at[pl.ds((c * NS + p) * (N * H) + off, L)],
                red_v.at[pl.ds(p * 2560, L)], rsem,
            ).wait()

        @plsc.parallel_loop(0, L // 16, unroll=2)
        def rb(j):
            acc = red_v[pl.ds(j * 16, 16)]
            for p in range(1, NS):
                acc = acc + red_v[pl.ds(p * 2560 + j * 16, 16)]
            invden_v[pl.ds(j * 16, 16)] = 1.0 / acc

        pltpu.sync_copy(
            invden_v.at[pl.ds(0, L)], inv_sp.at[pl.ds(off, L)]
        )

    @pl.when(s < NS - 1)
    def _():
        _reduce(s * 2496, 2496)

    @pl.when(s == NS - 1)
    def _():
        _reduce((NS - 1) * 2496, 2560)

    plsc.subcore_barrier()
    pltpu.sync_copy(inv_sp, invden_v)

    def fire_in(ci, b):
        ch = s + ci * NS

        @pl.when(ch < NCH)
        def _():
            pltpu.async_copy(edc_hbm.at[c, ch], edc_v.at[b], isems[b])
            pltpu.async_copy(ex_hbm.at[c, ch], ex_v.at[b], isems[b])

    def process(ci, b):
        ch = s + ci * NS

        @pl.when(ch < NCH)
        def _():
            pltpu.make_async_copy(
                edc_hbm.at[c, ch], edc_v.at[b], isems[b]
            ).wait()
            pltpu.make_async_copy(
                ex_hbm.at[c, ch], ex_v.at[b], isems[b]
            ).wait()

            @plsc.parallel_loop(0, CH // 16, unroll=2)
            def ab(g):
                dst16 = edc_v[b, 1, pl.ds(g * 16, 16)]
                for h in range(H):
                    ivd = plsc.load_gather(invden_v, [dst16 * H + h])
                    a = ex_v[b, h, pl.ds(g * 16, 16)] * ivd
                    plsc.store_scatter(
                        alpha_v, [(lanes + g * 16) * H + h + b * (CH * H)], a
                    )

            @pl.when(c == 0)
            def _():
                pltpu.async_copy(
                    alpha_v.at[pl.ds(b * (CH * H), CH * H)],
                    alpha0_hbm.at[pl.ds(ch * (CH * H), CH * H)], oasems[b],
                )

            @pl.when(c == 1)
            def _():
                pltpu.async_copy(
                    alpha_v.at[pl.ds(b * (CH * H), CH * H)],
                    alpha1_hbm.at[pl.ds(ch * (CH * H), CH * H)], oasems[b],
                )

    def drain_out(ci, b):
        ch = s + ci * NS

        @pl.when(ch < NCH)
        def _():
            pltpu.make_async_copy(
                alpha_v.at[pl.ds(b * (CH * H), CH * H)],
                alpha0_hbm.at[pl.ds(ch * (CH * H), CH * H)], oasems[b],
            ).wait()

    fire_in(0, 0)

    def pair_body(i2, carry):
        for b in (0, 1):
            ci = i2 * 2 + b
            fire_in(ci + 1, 1 - b)

            @pl.when(ci >= 2)
            def _():
                drain_out(ci - 2, b)

            process(ci, b)
        return carry

    lax.fori_loop(0, (CPB + 2) // 2, pair_body, 0)


C2 = 128
NCH2 = E // C2
CPB2 = -(-NCH2 // NS)


@functools.partial(
    pl.kernel,
    out_type=jax.ShapeDtypeStruct((2, N, HD), _f32),
    mesh=_sc_mesh(),
    scratch_types=[
        pltpu.VMEM((2, 2, C2), _i32),
        pltpu.VMEM((2, C2 * H), _f32),
        pltpu.VMEM((2, C2), _i32),
        pltpu.VMEM((2, C2), _i32),
        pltpu.VMEM((2, C2, HD), _f32),
        pltpu.VMEM((ZR, HD), _f32),
        pltpu.VMEM_SHARED((N, HD), _f32),
        pltpu.SemaphoreType.DMA,
        pltpu.SemaphoreType.DMA,
        pltpu.SemaphoreType.DMA,
        pltpu.SemaphoreType.DMA,
        pltpu.SemaphoreType.DMA,
        pltpu.SemaphoreType.DMA,
    ],
    compiler_params=pltpu.CompilerParams(needs_layout_passes=False),
)
def _phase2b(edc_hbm, alpha0_hbm, alpha1_hbm, feat_hbm, rst_hbm,
             edc_v, alpha_v, gidx_v, sidx_v, rows_v, zbuf_v, rst_sp,
             gsem0, gsem1, ssem0, ssem1, asem0, asem1):
    c = lax.axis_index("c")
    s = lax.axis_index("s")
    zeros16 = jnp.zeros((16,), _f32)
    gsems = (gsem0, gsem1)
    ssems = (ssem0, ssem1)
    asems = (asem0, asem1)

    def zb(i, carry):
        zbuf_v[i // 8, pl.ds((i % 8) * 16, 16)] = zeros16
        return carry

    lax.fori_loop(0, ZR * 8, zb, 0)
    for k in range(RB // ZR):
        pltpu.sync_copy(zbuf_v, rst_sp.at[pl.ds(s * RB + k * ZR, ZR)])

    @pl.when(s == NS - 1)
    def _():
        pltpu.sync_copy(zbuf_v.at[pl.ds(0, 16)], rst_sp.at[pl.ds(N - 16, 16)])

    plsc.subcore_barrier()

    def stage(ci, b):
        ch = s + ci * NS

        @pl.when(ch < NCH2)
        def _():
            pltpu.sync_copy(
                edc_hbm.at[c, ch // 4, :, pl.ds((ch % 4) * C2, C2)],
                edc_v.at[b],
            )

            @pl.when(c == 0)
            def _():
                pltpu.async_copy(
                    alpha0_hbm.at[pl.ds(ch * (C2 * H), C2 * H)],
                    alpha_v.at[b], asems[b],
                )

            @pl.when(c == 1)
            def _():
                pltpu.async_copy(
                    alpha1_hbm.at[pl.ds(ch * (C2 * H), C2 * H)],
                    alpha_v.at[b], asems[b],
                )

            @plsc.parallel_loop(0, C2 // 16, unroll=2)
            def ib(g):
                src16 = edc_v[b, 0, pl.ds(g * 16, 16)]
                dst16 = edc_v[b, 1, pl.ds(g * 16, 16)]
                gidx_v[b, pl.ds(g * 16, 16)] = src16 + c * N
                sidx_v[b, pl.ds(g * 16, 16)] = dst16
            pltpu.async_copy(
                feat_hbm.at[gidx_v.at[b]], rows_v.at[b], gsems[b]
            )

    def process(ci, b):
        ch = s + ci * NS

        @pl.when(ch < NCH2)
        def _():
            pltpu.make_async_copy(
                feat_hbm.at[gidx_v.at[b]], rows_v.at[b], gsems[b]
            ).wait()
            pltpu.make_async_copy(
                alpha0_hbm.at[pl.ds(ch * (C2 * H), C2 * H)],
                alpha_v.at[b], asems[b],
            ).wait()

            @plsc.parallel_loop(0, C2 // 4, unroll=4)
            def sb(g):
                a16 = alpha_v[b, pl.ds(g * 16, 16)]
                for k in range(4):
                    for h in range(H):
                        av = jnp.full((16,), a16[k * H + h], _f32)
                        for q in range(2):
                            off = h * D + q * 16
                            rows_v[b, g * 4 + k, pl.ds(off, 16)] = (
                                rows_v[b, g * 4 + k, pl.ds(off, 16)] * av
                            )

            pltpu.async_copy(
                rows_v.at[b], rst_sp.at[sidx_v.at[b]], ssems[b], add=True
            )

    def drain_scatter(ci, b):
        ch = s + ci * NS

        @pl.when(ch < NCH2)
        def _():
            pltpu.make_async_copy(
                rows_v.at[b], rst_sp.at[sidx_v.at[b]], ssems[b]
            ).wait()

    stage(0, 0)

    def pair_body(i2, carry):
        for b in (0, 1):
            ci = i2 * 2 + b
            bn = 1 - b

            @pl.when(ci >= 1)
            def _():
                drain_scatter(ci - 1, bn)

            stage(ci + 1, bn)
            process(ci, b)
        return carry

    lax.fori_loop(0, (CPB2 + 1) // 2, pair_body, 0)

    plsc.subcore_barrier()

    @pl.when(s < NS - 1)
    def _():
        pltpu.sync_copy(
            rst_sp.at[pl.ds(s * RB, RB)], rst_hbm.at[c, pl.ds(s * RB, RB)]
        )

    @pl.when(s == NS - 1)
    def _():
        pltpu.sync_copy(
            rst_sp.at[pl.ds((NS - 1) * RB, N - (NS - 1) * RB)],
            rst_hbm.at[c, pl.ds((NS - 1) * RB, N - (NS - 1) * RB)],
        )


def _final_body(rst_ref, x_ref, h0_ref, h1_ref):
    r = rst_ref[...] + x_ref[...]
    out = jnp.where(r > 0, r, jnp.exp(jnp.minimum(r, 0.0)) - 1.0)
    h0_ref[...] = out[0]
    h1_ref[...] = out[1]


def _final(rst, xs):
    return pl.pallas_call(
        _final_body,
        grid=(10,),
        in_specs=[
            pl.BlockSpec((2, 1000, HD), lambda i: (0, i, 0)),
            pl.BlockSpec((2, 1000, HD), lambda i: (0, i, 0)),
        ],
        out_specs=[
            pl.BlockSpec((1000, HD), lambda i: (i, 0)),
            pl.BlockSpec((1000, HD), lambda i: (i, 0)),
        ],
        out_shape=[
            jax.ShapeDtypeStruct((N, HD), _f32),
            jax.ShapeDtypeStruct((N, HD), _f32),
        ],
    )(rst, xs)


def kernel(x0, x1, edge_index0, edge_index1, W0, al0, ar0, W1, al1, ar1):
    xs = jnp.stack([x0, x1])
    Ws = jnp.stack([W0, W1])

    eye = np.eye(H, dtype=np.float32)

    def mk_alr(al, ar):
        a_el = (al[:, :, None] * eye[:, None, :]).reshape(HD, H)
        a_er = (ar[:, :, None] * eye[:, None, :]).reshape(HD, H)
        return jnp.concatenate([a_el, a_er], axis=1)

    Alrs = jnp.stack([mk_alr(al0, ar0), mk_alr(al1, ar1)])

    edc = (
        jnp.stack([edge_index0, edge_index1])
        .reshape(2, 2, NCH, CH)
        .transpose(0, 2, 1, 3)
    )

    feat, elr = _prep(xs, Ws, Alrs)
    elr_flat = elr.reshape(2 * N * 2 * H)

    ex, parts = _phase1(elr_flat, edc)
    alpha0, alpha1 = _phase2a(edc, ex, parts)
    rst = _phase2b(edc, alpha0, alpha1, feat)

    h0, h1 = _final(rst, xs)
    return (h0, h1, alpha0.reshape(E, H, 1), alpha1.reshape(E, H, 1))


# speedup vs baseline: 1.0008x; 1.0008x over previous
import functools

import numpy as np

import jax
import jax.numpy as jnp
from jax import lax
from jax.experimental import pallas as pl
from jax.experimental.pallas import tpu as pltpu
from jax.experimental.pallas import tpu_sc as plsc

N = 10000
E = 320000
H = 4
D = 32
DIM = 128
HD = H * D

NC = 2
NS = 16
CH = 512
NCH = E // CH
CPB = -(-NCH // NS)
RB = 624
ZR = 48

_f32 = jnp.float32
_i32 = jnp.int32


def _prep_body(x_ref, w_ref, alr_ref, feat_ref, elr_ref):
    x = x_ref[0]
    feat = jnp.dot(x, w_ref[0], preferred_element_type=_f32)
    elr = jnp.dot(feat, alr_ref[0], preferred_element_type=_f32)
    feat_ref[...] = feat
    elr_ref[...] = elr[None]


def _prep(xs, Ws, Alrs):
    return pl.pallas_call(
        _prep_body,
        grid=(2, 10),
        in_specs=[
            pl.BlockSpec((1, 1000, DIM), lambda l, i: (l, i, 0)),
            pl.BlockSpec((1, DIM, HD), lambda l, i: (l, 0, 0)),
            pl.BlockSpec((1, HD, 2 * H), lambda l, i: (l, 0, 0)),
        ],
        out_specs=[
            pl.BlockSpec((1000, HD), lambda l, i: (l * 10 + i, 0)),
            pl.BlockSpec((1, 1000, 2 * H), lambda l, i: (l, i, 0)),
        ],
        out_shape=[
            jax.ShapeDtypeStruct((2 * N, HD), _f32),
            jax.ShapeDtypeStruct((2, N, 2 * H), _f32),
        ],
    )(xs, Ws, Alrs)


def _sc_mesh():
    return plsc.VectorSubcoreMesh(core_axis_name="c", subcore_axis_name="s")


@functools.partial(
    pl.kernel,
    out_type=(
        jax.ShapeDtypeStruct((2, NCH, H, CH), _f32),
        jax.ShapeDtypeStruct((2 * NS * N * H,), _f32),
    ),
    mesh=_sc_mesh(),
    scratch_types=[
        pltpu.VMEM((N * 2 * H,), _f32),
        pltpu.VMEM((N * H,), _f32),
        pltpu.VMEM((2, 2, CH), _i32),
        pltpu.VMEM((2, H, CH), _f32),
        pltpu.SemaphoreType.DMA,
        pltpu.SemaphoreType.DMA,
        pltpu.SemaphoreType.DMA,
        pltpu.SemaphoreType.DMA,
    ],
    compiler_params=pltpu.CompilerParams(needs_layout_passes=False),
)
def _phase1(elr_hbm, edc_hbm, ex_hbm, part_hbm, elr_v, den_v, edc_v, ex_v,
            iesem0, iesem1, oxsem0, oxsem1):
    c = lax.axis_index("c")
    s = lax.axis_index("s")
    zeros16 = jnp.zeros((16,), _f32)
    iesems = (iesem0, iesem1)
    oxsems = (oxsem0, oxsem1)

    pltpu.sync_copy(elr_hbm.at[pl.ds(c * (N * 2 * H), N * 2 * H)], elr_v)

    @plsc.parallel_loop(0, (N * H) // 16, unroll=8)
    def zb(i):
        den_v[pl.ds(i * 16, 16)] = zeros16

    def fire_in(ci, b):
        ch = s + ci * NS

        @pl.when(ch < NCH)
        def _():
            pltpu.async_copy(edc_hbm.at[c, ch], edc_v.at[b], iesems[b])

    def process(ci, b):
        ch = s + ci * NS

        @pl.when(ch < NCH)
        def _():
            pltpu.make_async_copy(
                edc_hbm.at[c, ch], edc_v.at[b], iesems[b]
            ).wait()

            @plsc.parallel_loop(0, CH // 16, unroll=2)
            def grp(g):
                src16 = edc_v[b, 0, pl.ds(g * 16, 16)]
                dst16 = edc_v[b, 1, pl.ds(g * 16, 16)]
                for h in range(H):
                    el = plsc.load_gather(elr_v, [src16 * (2 * H) + h])
                    er = plsc.load_gather(elr_v, [dst16 * (2 * H) + (H + h)])
                    e = el + er
                    e = jnp.where(e >= 0, e, 0.2 * e)
                    ex = jnp.exp(e)
                    ex_v[b, h, pl.ds(g * 16, 16)] = ex
                    plsc.addupdate_scatter(den_v, [dst16 * H + h], ex)
            pltpu.async_copy(ex_v.at[b], ex_hbm.at[c, ch], oxsems[b])

    def drain_out(ci, b):
        ch = s + ci * NS

        @pl.when(ch < NCH)
        def _():
            pltpu.make_async_copy(
                ex_v.at[b], ex_hbm.at[c, ch], oxsems[b]
            ).wait()

    fire_in(0, 0)

    def pair_body(i2, carry):
        for b in (0, 1):
            ci = i2 * 2 + b
            fire_in(ci + 1, 1 - b)

            @pl.when(ci >= 2)
            def _():
                drain_out(ci - 2, b)

            process(ci, b)
        return carry

    lax.fori_loop(0, (CPB + 2) // 2, pair_body, 0)
    pltpu.sync_copy(den_v, part_hbm.at[pl.ds((c * NS + s) * (N * H), N * H)])


@functools.partial(
    pl.kernel,
    out_type=(
        jax.ShapeDtypeStruct((E * H,), _f32),
        jax.ShapeDtypeStruct((E * H,), _f32),
    ),
    mesh=_sc_mesh(),
    scratch_types=[
        pltpu.VMEM((N * H,), _f32),
        pltpu.VMEM((NS * 2560,), _f32),
        pltpu.VMEM((2, 2, CH), _i32),
        pltpu.VMEM((2, H, CH), _f32),
        pltpu.VMEM((2 * CH * H,), _f32),
        pltpu.VMEM_SHARED((N * H,), _f32),
        pltpu.SemaphoreType.DMA,
        pltpu.SemaphoreType.DMA,
        pltpu.SemaphoreType.DMA,
        pltpu.SemaphoreType.DMA,
        pltpu.SemaphoreType.DMA,
    ],
    compiler_params=pltpu.CompilerParams(needs_layout_passes=False),
)
def _phase2a(edc_hbm, ex_hbm, part_hbm, alpha0_hbm, alpha1_hbm,
             invden_v, red_v, edc_v, ex_v, alpha_v, inv_sp,
             isem0, isem1, oasem0, oasem1, rsem):
    c = lax.axis_index("c")
    s = lax.axis_index("s")
    lanes = lax.iota(_i32, 16)
    isems = (isem0, isem1)
    oasems = (oasem0, oasem1)

    def _reduce(off, L):
        for p in range(NS):
            pltpu.async_copy(
                part_hbm.at[pl.ds((c * NS + p) * (N * H) + off, L)],
                red_v.at[pl.ds(p * 2560, L)], rsem,
            )
        for p in range(NS):
            pltpu.make_async_copy(
                part_hbm.at[pl.ds((c * NS + p) * (N * H) + off, L)],
                red_v.at[pl.ds(p * 2560, L)], rsem,
            ).wait()

        @plsc.parallel_loop(0, L // 16, unroll=2)
        def rb(j):
            acc = red_v[pl.ds(j * 16, 16)]
            for p in range(1, NS):
                acc = acc + red_v[pl.ds(p * 2560 + j * 16, 16)]
            invden_v[pl.ds(j * 16, 16)] = 1.0 / acc

        pltpu.sync_copy(
            invden_v.at[pl.ds(0, L)], inv_sp.at[pl.ds(off, L)]
        )

    @pl.when(s < NS - 1)
    def _():
        _reduce(s * 2496, 2496)

    @pl.when(s == NS - 1)
    def _():
        _reduce((NS - 1) * 2496, 2560)

    plsc.subcore_barrier()
    pltpu.sync_copy(inv_sp, invden_v)

    def fire_in(ci, b):
        ch = s + ci * NS

        @pl.when(ch < NCH)
        def _():
            pltpu.async_copy(edc_hbm.at[c, ch], edc_v.at[b], isems[b])
            pltpu.async_copy(ex_hbm.at[c, ch], ex_v.at[b], isems[b])

    def process(ci, b):
        ch = s + ci * NS

        @pl.when(ch < NCH)
        def _():
            pltpu.make_async_copy(
                edc_hbm.at[c, ch], edc_v.at[b], isems[b]
            ).wait()
            pltpu.make_async_copy(
                ex_hbm.at[c, ch], ex_v.at[b], isems[b]
            ).wait()

            @plsc.parallel_loop(0, CH // 16, unroll=2)
            def ab(g):
                dst16 = edc_v[b, 1, pl.ds(g * 16, 16)]
                for h in range(H):
                    ivd = plsc.load_gather(invden_v, [dst16 * H + h])
                    a = ex_v[b, h, pl.ds(g * 16, 16)] * ivd
                    plsc.store_scatter(
                        alpha_v, [(lanes + g * 16) * H + h + b * (CH * H)], a
                    )

            @pl.when(c == 0)
            def _():
                pltpu.async_copy(
                    alpha_v.at[pl.ds(b * (CH * H), CH * H)],
                    alpha0_hbm.at[pl.ds(ch * (CH * H), CH * H)], oasems[b],
                )

            @pl.when(c == 1)
            def _():
                pltpu.async_copy(
                    alpha_v.at[pl.ds(b * (CH * H), CH * H)],
                    alpha1_hbm.at[pl.ds(ch * (CH * H), CH * H)], oasems[b],
                )

    def drain_out(ci, b):
        ch = s + ci * NS

        @pl.when(ch < NCH)
        def _():
            pltpu.make_async_copy(
                alpha_v.at[pl.ds(b * (CH * H), CH * H)],
                alpha0_hbm.at[pl.ds(ch * (CH * H), CH * H)], oasems[b],
            ).wait()

    fire_in(0, 0)

    def pair_body(i2, carry):
        for b in (0, 1):
            ci = i2 * 2 + b
            fire_in(ci + 1, 1 - b)

            @pl.when(ci >= 2)
            def _():
                drain_out(ci - 2, b)

            process(ci, b)
        return carry

    lax.fori_loop(0, (CPB + 2) // 2, pair_body, 0)


C2 = 128
NCH2 = E // C2
CPB2 = -(-NCH2 // NS)


@functools.partial(
    pl.kernel,
    out_type=jax.ShapeDtypeStruct((2, N, HD), _f32),
    mesh=_sc_mesh(),
    scratch_types=[
        pltpu.VMEM((2, 2, C2), _i32),
        pltpu.VMEM((2, C2 * H), _f32),
        pltpu.VMEM((2, C2), _i32),
        pltpu.VMEM((2, C2), _i32),
        pltpu.VMEM((2, C2, HD), _f32),
        pltpu.VMEM((ZR, HD), _f32),
        pltpu.VMEM_SHARED((N, HD), _f32),
        pltpu.SemaphoreType.DMA,
        pltpu.SemaphoreType.DMA,
        pltpu.SemaphoreType.DMA,
        pltpu.SemaphoreType.DMA,
        pltpu.SemaphoreType.DMA,
        pltpu.SemaphoreType.DMA,
    ],
    compiler_params=pltpu.CompilerParams(needs_layout_passes=False),
)
def _phase2b(edc_hbm, alpha0_hbm, alpha1_hbm, feat_hbm, rst_hbm,
             edc_v, alpha_v, gidx_v, sidx_v, rows_v, zbuf_v, rst_sp,
             gsem0, gsem1, ssem0, ssem1, asem0, asem1):
    c = lax.axis_index("c")
    s = lax.axis_index("s")
    zeros16 = jnp.zeros((16,), _f32)
    gsems = (gsem0, gsem1)
    ssems = (ssem0, ssem1)
    asems = (asem0, asem1)

    def zb(i, carry):
        zbuf_v[i // 8, pl.ds((i % 8) * 16, 16)] = zeros16
        return carry

    lax.fori_loop(0, ZR * 8, zb, 0)
    for k in range(RB // ZR):
        pltpu.sync_copy(zbuf_v, rst_sp.at[pl.ds(s * RB + k * ZR, ZR)])

    @pl.when(s == NS - 1)
    def _():
        pltpu.sync_copy(zbuf_v.at[pl.ds(0, 16)], rst_sp.at[pl.ds(N - 16, 16)])

    plsc.subcore_barrier()

    def stage(ci, b):
        ch = s + ci * NS

        @pl.when(ch < NCH2)
        def _():
            pltpu.sync_copy(
                edc_hbm.at[c, ch // 4, :, pl.ds((ch % 4) * C2, C2)],
                edc_v.at[b],
            )

            @pl.when(c == 0)
            def _():
                pltpu.async_copy(
                    alpha0_hbm.at[pl.ds(ch * (C2 * H), C2 * H)],
                    alpha_v.at[b], asems[b],
                )

            @pl.when(c == 1)
            def _():
                pltpu.async_copy(
                    alpha1_hbm.at[pl.ds(ch * (C2 * H), C2 * H)],
                    alpha_v.at[b], asems[b],
                )

            @plsc.parallel_loop(0, C2 // 16, unroll=2)
            def ib(g):
                src16 = edc_v[b, 0, pl.ds(g * 16, 16)]
                dst16 = edc_v[b, 1, pl.ds(g * 16, 16)]
                gidx_v[b, pl.ds(g * 16, 16)] = src16 + c * N
                sidx_v[b, pl.ds(g * 16, 16)] = dst16
            pltpu.async_copy(
                feat_hbm.at[gidx_v.at[b]], rows_v.at[b], gsems[b]
            )

    def process(ci, b):
        ch = s + ci * NS

        @pl.when(ch < NCH2)
        def _():
            pltpu.make_async_copy(
                feat_hbm.at[gidx_v.at[b]], rows_v.at[b], gsems[b]
            ).wait()
            pltpu.make_async_copy(
                alpha0_hbm.at[pl.ds(ch * (C2 * H), C2 * H)],
                alpha_v.at[b], asems[b],
            ).wait()

            @plsc.parallel_loop(0, C2 // 4, unroll=2)
            def sb(g):
                a16 = alpha_v[b, pl.ds(g * 16, 16)]
                for k in range(4):
                    for h in range(H):
                        av = jnp.full((16,), a16[k * H + h], _f32)
                        for q in range(2):
                            off = h * D + q * 16
                            rows_v[b, g * 4 + k, pl.ds(off, 16)] = (
                                rows_v[b, g * 4 + k, pl.ds(off, 16)] * av
                            )

            pltpu.async_copy(
                rows_v.at[b], rst_sp.at[sidx_v.at[b]], ssems[b], add=True
            )

    def drain_scatter(ci, b):
        ch = s + ci * NS

        @pl.when(ch < NCH2)
        def _():
            pltpu.make_async_copy(
                rows_v.at[b], rst_sp.at[sidx_v.at[b]], ssems[b]
            ).wait()

    stage(0, 0)

    def pair_body(i2, carry):
        for b in (0, 1):
            ci = i2 * 2 + b
            bn = 1 - b

            @pl.when(ci >= 1)
            def _():
                drain_scatter(ci - 1, bn)

            stage(ci + 1, bn)
            process(ci, b)
        return carry

    lax.fori_loop(0, (CPB2 + 1) // 2, pair_body, 0)

    plsc.subcore_barrier()

    @pl.when(s < NS - 1)
    def _():
        pltpu.sync_copy(
            rst_sp.at[pl.ds(s * RB, RB)], rst_hbm.at[c, pl.ds(s * RB, RB)]
        )

    @pl.when(s == NS - 1)
    def _():
        pltpu.sync_copy(
            rst_sp.at[pl.ds((NS - 1) * RB, N - (NS - 1) * RB)],
            rst_hbm.at[c, pl.ds((NS - 1) * RB, N - (NS - 1) * RB)],
        )


def _final_body(rst_ref, x_ref, h0_ref, h1_ref):
    r = rst_ref[...] + x_ref[...]
    out = jnp.where(r > 0, r, jnp.exp(jnp.minimum(r, 0.0)) - 1.0)
    h0_ref[...] = out[0]
    h1_ref[...] = out[1]


def _final(rst, xs):
    return pl.pallas_call(
        _final_body,
        grid=(10,),
        in_specs=[
            pl.BlockSpec((2, 1000, HD), lambda i: (0, i, 0)),
            pl.BlockSpec((2, 1000, HD), lambda i: (0, i, 0)),
        ],
        out_specs=[
            pl.BlockSpec((1000, HD), lambda i: (i, 0)),
            pl.BlockSpec((1000, HD), lambda i: (i, 0)),
        ],
        out_shape=[
            jax.ShapeDtypeStruct((N, HD), _f32),
            jax.ShapeDtypeStruct((N, HD), _f32),
        ],
    )(rst, xs)


def kernel(x0, x1, edge_index0, edge_index1, W0, al0, ar0, W1, al1, ar1):
    xs = jnp.stack([x0, x1])
    Ws = jnp.stack([W0, W1])

    eye = np.eye(H, dtype=np.float32)

    def mk_alr(al, ar):
        a_el = (al[:, :, None] * eye[:, None, :]).reshape(HD, H)
        a_er = (ar[:, :, None] * eye[:, None, :]).reshape(HD, H)
        return jnp.concatenate([a_el, a_er], axis=1)

    Alrs = jnp.stack([mk_alr(al0, ar0), mk_alr(al1, ar1)])

    edc = (
        jnp.stack([edge_index0, edge_index1])
        .reshape(2, 2, NCH, CH)
        .transpose(0, 2, 1, 3)
    )

    feat, elr = _prep(xs, Ws, Alrs)
    elr_flat = elr.reshape(2 * N * 2 * H)

    ex, parts = _phase1(elr_flat, edc)
    alpha0, alpha1 = _phase2a(edc, ex, parts)
    rst = _phase2b(edc, alpha0, alpha1, feat)

    h0, h1 = _final(rst, xs)
    return (h0, h1, alpha0.reshape(E, H, 1), alpha1.reshape(E, H, 1))


# speedup vs baseline: 1.0122x; 1.0114x over previous
import functools

import numpy as np

import jax
import jax.numpy as jnp
from jax import lax
from jax.experimental import pallas as pl
from jax.experimental.pallas import tpu as pltpu
from jax.experimental.pallas import tpu_sc as plsc

N = 10000
E = 320000
H = 4
D = 32
DIM = 128
HD = H * D

NC = 2
NS = 16
CH = 512
NCH = E // CH
CPB = -(-NCH // NS)
RB = 624
ZR = 48

_f32 = jnp.float32
_i32 = jnp.int32


def _prep_body(x_ref, w_ref, alr_ref, feat_ref, elr_ref):
    x = x_ref[0]
    feat = jnp.dot(x, w_ref[0], preferred_element_type=_f32)
    elr = jnp.dot(feat, alr_ref[0], preferred_element_type=_f32)
    feat_ref[...] = feat
    elr_ref[...] = elr[None]


def _prep(xs, Ws, Alrs):
    return pl.pallas_call(
        _prep_body,
        grid=(2, 10),
        in_specs=[
            pl.BlockSpec((1, 1000, DIM), lambda l, i: (l, i, 0)),
            pl.BlockSpec((1, DIM, HD), lambda l, i: (l, 0, 0)),
            pl.BlockSpec((1, HD, 2 * H), lambda l, i: (l, 0, 0)),
        ],
        out_specs=[
            pl.BlockSpec((1000, HD), lambda l, i: (l * 10 + i, 0)),
            pl.BlockSpec((1, 1000, 2 * H), lambda l, i: (l, i, 0)),
        ],
        out_shape=[
            jax.ShapeDtypeStruct((2 * N, HD), _f32),
            jax.ShapeDtypeStruct((2, N, 2 * H), _f32),
        ],
    )(xs, Ws, Alrs)


def _sc_mesh():
    return plsc.VectorSubcoreMesh(core_axis_name="c", subcore_axis_name="s")


@functools.partial(
    pl.kernel,
    out_type=(
        jax.ShapeDtypeStruct((2, NCH, H, CH), _f32),
        jax.ShapeDtypeStruct((2 * NS * N * H,), _f32),
        jax.ShapeDtypeStruct((E * H,), _f32),
        jax.ShapeDtypeStruct((E * H,), _f32),
    ),
    mesh=_sc_mesh(),
    scratch_types=[
        pltpu.VMEM((N * 2 * H,), _f32),
        pltpu.VMEM((N * H,), _f32),
        pltpu.VMEM((2, 2, CH), _i32),
        pltpu.VMEM((2, H, CH), _f32),
        pltpu.VMEM_SHARED((N * H,), _f32),
        pltpu.SemaphoreType.DMA,
        pltpu.SemaphoreType.DMA,
        pltpu.SemaphoreType.DMA,
        pltpu.SemaphoreType.DMA,
        pltpu.SemaphoreType.DMA,
    ],
    compiler_params=pltpu.CompilerParams(needs_layout_passes=False),
)
def _phase1(elr_hbm, edc_hbm, ex_hbm, part_hbm, alpha0_hbm, alpha1_hbm,
            elr_v, den_v, edc_v, ex_v, inv_sp,
            iesem0, iesem1, oxsem0, oxsem1, rsem):
    c = lax.axis_index("c")
    s = lax.axis_index("s")
    lanes = lax.iota(_i32, 16)
    zeros16 = jnp.zeros((16,), _f32)
    iesems = (iesem0, iesem1)
    oxsems = (oxsem0, oxsem1)

    pltpu.sync_copy(elr_hbm.at[pl.ds(c * (N * 2 * H), N * 2 * H)], elr_v)

    @plsc.parallel_loop(0, (N * H) // 16, unroll=8)
    def zb(i):
        den_v[pl.ds(i * 16, 16)] = zeros16

    def fire_in(ci, b):
        ch = s + ci * NS

        @pl.when(ch < NCH)
        def _():
            pltpu.async_copy(edc_hbm.at[c, ch], edc_v.at[b], iesems[b])

    def process(ci, b):
        ch = s + ci * NS

        @pl.when(ch < NCH)
        def _():
            pltpu.make_async_copy(
                edc_hbm.at[c, ch], edc_v.at[b], iesems[b]
            ).wait()

            @plsc.parallel_loop(0, CH // 16, unroll=2)
            def grp(g):
                src16 = edc_v[b, 0, pl.ds(g * 16, 16)]
                dst16 = edc_v[b, 1, pl.ds(g * 16, 16)]
                for h in range(H):
                    el = plsc.load_gather(elr_v, [src16 * (2 * H) + h])
                    er = plsc.load_gather(elr_v, [dst16 * (2 * H) + (H + h)])
                    e = el + er
                    e = jnp.where(e >= 0, e, 0.2 * e)
                    ex = jnp.exp(e)
                    ex_v[b, h, pl.ds(g * 16, 16)] = ex
                    plsc.addupdate_scatter(den_v, [dst16 * H + h], ex)

            pltpu.async_copy(ex_v.at[b], ex_hbm.at[c, ch], oxsems[b])

    def drain_out(ci, b):
        ch = s + ci * NS

        @pl.when(ch < NCH)
        def _():
            pltpu.make_async_copy(
                ex_v.at[b], ex_hbm.at[c, ch], oxsems[b]
            ).wait()

    fire_in(0, 0)

    def pair_body(i2, carry):
        for b in (0, 1):
            ci = i2 * 2 + b
            fire_in(ci + 1, 1 - b)

            @pl.when(ci >= 2)
            def _():
                drain_out(ci - 2, b)

            process(ci, b)
        return carry

    lax.fori_loop(0, (CPB + 2) // 2, pair_body, 0)
    pltpu.sync_copy(den_v, part_hbm.at[pl.ds((c * NS + s) * (N * H), N * H)])
    plsc.subcore_barrier()

    def _reduce(off, L):
        for p in range(NS):
            pltpu.async_copy(
                part_hbm.at[pl.ds((c * NS + p) * (N * H) + off, L)],
                elr_v.at[pl.ds(p * 2560, L)], rsem,
            )
        for p in range(NS):
            pltpu.make_async_copy(
                part_hbm.at[pl.ds((c * NS + p) * (N * H) + off, L)],
                elr_v.at[pl.ds(p * 2560, L)], rsem,
            ).wait()

        @plsc.parallel_loop(0, L // 16, unroll=2)
        def rb(j):
            acc = elr_v[pl.ds(j * 16, 16)]
            for p in range(1, NS):
                acc = acc + elr_v[pl.ds(p * 2560 + j * 16, 16)]
            den_v[pl.ds(j * 16, 16)] = 1.0 / acc

        pltpu.sync_copy(den_v.at[pl.ds(0, L)], inv_sp.at[pl.ds(off, L)])

    @pl.when(s < NS - 1)
    def _():
        _reduce(s * 2496, 2496)

    @pl.when(s == NS - 1)
    def _():
        _reduce((NS - 1) * 2496, 2560)

    plsc.subcore_barrier()
    pltpu.sync_copy(inv_sp, den_v)

    AOFF = NS * 2560

    def fire_in2(ci, b):
        ch = s + ci * NS

        @pl.when(ch < NCH)
        def _():
            pltpu.async_copy(edc_hbm.at[c, ch], edc_v.at[b], iesems[b])
            pltpu.async_copy(ex_hbm.at[c, ch], ex_v.at[b], iesems[b])

    def process2(ci, b):
        ch = s + ci * NS

        @pl.when(ch < NCH)
        def _():
            pltpu.make_async_copy(
                edc_hbm.at[c, ch], edc_v.at[b], iesems[b]
            ).wait()
            pltpu.make_async_copy(
                ex_hbm.at[c, ch], ex_v.at[b], iesems[b]
            ).wait()

            @plsc.parallel_loop(0, CH // 16, unroll=2)
            def ab(g):
                dst16 = edc_v[b, 1, pl.ds(g * 16, 16)]
                for h in range(H):
                    ivd = plsc.load_gather(den_v, [dst16 * H + h])
                    a = ex_v[b, h, pl.ds(g * 16, 16)] * ivd
                    plsc.store_scatter(
                        elr_v,
                        [(lanes + g * 16) * H + h + (AOFF + b * (CH * H))],
                        a,
                    )

            @pl.when(c == 0)
            def _():
                pltpu.async_copy(
                    elr_v.at[pl.ds(AOFF + b * (CH * H), CH * H)],
                    alpha0_hbm.at[pl.ds(ch * (CH * H), CH * H)], oxsems[b],
                )

            @pl.when(c == 1)
            def _():
                pltpu.async_copy(
                    elr_v.at[pl.ds(AOFF + b * (CH * H), CH * H)],
                    alpha1_hbm.at[pl.ds(ch * (CH * H), CH * H)], oxsems[b],
                )

    def drain_out2(ci, b):
        ch = s + ci * NS

        @pl.when(ch < NCH)
        def _():
            pltpu.make_async_copy(
                elr_v.at[pl.ds(AOFF + b * (CH * H), CH * H)],
                alpha0_hbm.at[pl.ds(ch * (CH * H), CH * H)], oxsems[b],
            ).wait()

    fire_in2(0, 0)

    def pair_body2(i2, carry):
        for b in (0, 1):
            ci = i2 * 2 + b
            fire_in2(ci + 1, 1 - b)

            @pl.when(ci >= 2)
            def _():
                drain_out2(ci - 2, b)

            process2(ci, b)
        return carry

    lax.fori_loop(0, (CPB + 2) // 2, pair_body2, 0)


C2 = 128
NCH2 = E // C2
CPB2 = -(-NCH2 // NS)


@functools.partial(
    pl.kernel,
    out_type=jax.ShapeDtypeStruct((2, N, HD), _f32),
    mesh=_sc_mesh(),
    scratch_types=[
        pltpu.VMEM((2, 2, C2), _i32),
        pltpu.VMEM((2, C2 * H), _f32),
        pltpu.VMEM((2, C2), _i32),
        pltpu.VMEM((2, C2), _i32),
        pltpu.VMEM((2, C2, HD), _f32),
        pltpu.VMEM((ZR, HD), _f32),
        pltpu.VMEM_SHARED((N, HD), _f32),
        pltpu.SemaphoreType.DMA,
        pltpu.SemaphoreType.DMA,
        pltpu.SemaphoreType.DMA,
        pltpu.SemaphoreType.DMA,
        pltpu.SemaphoreType.DMA,
        pltpu.SemaphoreType.DMA,
    ],
    compiler_params=pltpu.CompilerParams(needs_layout_passes=False),
)
def _phase2b(edc_hbm, alpha0_hbm, alpha1_hbm, feat_hbm, rst_hbm,
             edc_v, alpha_v, gidx_v, sidx_v, rows_v, zbuf_v, rst_sp,
             gsem0, gsem1, ssem0, ssem1, asem0, asem1):
    c = lax.axis_index("c")
    s = lax.axis_index("s")
    zeros16 = jnp.zeros((16,), _f32)
    gsems = (gsem0, gsem1)
    ssems = (ssem0, ssem1)
    asems = (asem0, asem1)

    def zb(i, carry):
        zbuf_v[i // 8, pl.ds((i % 8) * 16, 16)] = zeros16
        return carry

    lax.fori_loop(0, ZR * 8, zb, 0)
    for k in range(RB // ZR):
        pltpu.sync_copy(zbuf_v, rst_sp.at[pl.ds(s * RB + k * ZR, ZR)])

    @pl.when(s == NS - 1)
    def _():
        pltpu.sync_copy(zbuf_v.at[pl.ds(0, 16)], rst_sp.at[pl.ds(N - 16, 16)])

    plsc.subcore_barrier()

    def stage(ci, b):
        ch = s + ci * NS

        @pl.when(ch < NCH2)
        def _():
            pltpu.sync_copy(
                edc_hbm.at[c, ch // 4, :, pl.ds((ch % 4) * C2, C2)],
                edc_v.at[b],
            )

            @pl.when(c == 0)
            def _():
                pltpu.async_copy(
                    alpha0_hbm.at[pl.ds(ch * (C2 * H), C2 * H)],
                    alpha_v.at[b], asems[b],
                )

            @pl.when(c == 1)
            def _():
                pltpu.async_copy(
                    alpha1_hbm.at[pl.ds(ch * (C2 * H), C2 * H)],
                    alpha_v.at[b], asems[b],
                )

            @plsc.parallel_loop(0, C2 // 16, unroll=2)
            def ib(g):
                src16 = edc_v[b, 0, pl.ds(g * 16, 16)]
                dst16 = edc_v[b, 1, pl.ds(g * 16, 16)]
                gidx_v[b, pl.ds(g * 16, 16)] = src16 + c * N
                sidx_v[b, pl.ds(g * 16, 16)] = dst16
            pltpu.async_copy(
                feat_hbm.at[gidx_v.at[b]], rows_v.at[b], gsems[b]
            )

    def process(ci, b):
        ch = s + ci * NS

        @pl.when(ch < NCH2)
        def _():
            pltpu.make_async_copy(
                feat_hbm.at[gidx_v.at[b]], rows_v.at[b], gsems[b]
            ).wait()
            pltpu.make_async_copy(
                alpha0_hbm.at[pl.ds(ch * (C2 * H), C2 * H)],
                alpha_v.at[b], asems[b],
            ).wait()

            @plsc.parallel_loop(0, C2 // 4, unroll=2)
            def sb(g):
                a16 = alpha_v[b, pl.ds(g * 16, 16)]
                for k in range(4):
                    for h in range(H):
                        av = jnp.full((16,), a16[k * H + h], _f32)
                        for q in range(2):
                            off = h * D + q * 16
                            rows_v[b, g * 4 + k, pl.ds(off, 16)] = (
                                rows_v[b, g * 4 + k, pl.ds(off, 16)] * av
                            )

            pltpu.async_copy(
                rows_v.at[b], rst_sp.at[sidx_v.at[b]], ssems[b], add=True
            )

    def drain_scatter(ci, b):
        ch = s + ci * NS

        @pl.when(ch < NCH2)
        def _():
            pltpu.make_async_copy(
                rows_v.at[b], rst_sp.at[sidx_v.at[b]], ssems[b]
            ).wait()

    stage(0, 0)

    def pair_body(i2, carry):
        for b in (0, 1):
            ci = i2 * 2 + b
            bn = 1 - b

            @pl.when(ci >= 1)
            def _():
                drain_scatter(ci - 1, bn)

            stage(ci + 1, bn)
            process(ci, b)
        return carry

    lax.fori_loop(0, (CPB2 + 1) // 2, pair_body, 0)

    plsc.subcore_barrier()

    @pl.when(s < NS - 1)
    def _():
        pltpu.sync_copy(
            rst_sp.at[pl.ds(s * RB, RB)], rst_hbm.at[c, pl.ds(s * RB, RB)]
        )

    @pl.when(s == NS - 1)
    def _():
        pltpu.sync_copy(
            rst_sp.at[pl.ds((NS - 1) * RB, N - (NS - 1) * RB)],
            rst_hbm.at[c, pl.ds((NS - 1) * RB, N - (NS - 1) * RB)],
        )


def _final_body(rst_ref, x_ref, h0_ref, h1_ref):
    r = rst_ref[...] + x_ref[...]
    out = jnp.where(r > 0, r, jnp.exp(jnp.minimum(r, 0.0)) - 1.0)
    h0_ref[...] = out[0]
    h1_ref[...] = out[1]


def _final(rst, xs):
    return pl.pallas_call(
        _final_body,
        grid=(10,),
        in_specs=[
            pl.BlockSpec((2, 1000, HD), lambda i: (0, i, 0)),
            pl.BlockSpec((2, 1000, HD), lambda i: (0, i, 0)),
        ],
        out_specs=[
            pl.BlockSpec((1000, HD), lambda i: (i, 0)),
            pl.BlockSpec((1000, HD), lambda i: (i, 0)),
        ],
        out_shape=[
            jax.ShapeDtypeStruct((N, HD), _f32),
            jax.ShapeDtypeStruct((N, HD), _f32),
        ],
    )(rst, xs)


def kernel(x0, x1, edge_index0, edge_index1, W0, al0, ar0, W1, al1, ar1):
    xs = jnp.stack([x0, x1])
    Ws = jnp.stack([W0, W1])

    eye = np.eye(H, dtype=np.float32)

    def mk_alr(al, ar):
        a_el = (al[:, :, None] * eye[:, None, :]).reshape(HD, H)
        a_er = (ar[:, :, None] * eye[:, None, :]).reshape(HD, H)
        return jnp.concatenate([a_el, a_er], axis=1)

    Alrs = jnp.stack([mk_alr(al0, ar0), mk_alr(al1, ar1)])

    edc = (
        jnp.stack([edge_index0, edge_index1])
        .reshape(2, 2, NCH, CH)
        .transpose(0, 2, 1, 3)
    )

    feat, elr = _prep(xs, Ws, Alrs)
    elr_flat = elr.reshape(2 * N * 2 * H)

    ex, parts, alpha0, alpha1 = _phase1(elr_flat, edc)
    rst = _phase2b(edc, alpha0, alpha1, feat)

    h0, h1 = _final(rst, xs)
    return (h0, h1, alpha0.reshape(E, H, 1), alpha1.reshape(E, H, 1))
